# Initial kernel scaffold; baseline (speedup 1.0000x reference)
import jax
import jax.numpy as jnp
from jax.experimental import pallas as pl


def kernel(x_nchw, dw_weight, pw_weight, gamma, beta):
    raise NotImplementedError("write your pallas kernel here")


# speedup vs baseline: 1.1454x; 1.1454x over previous
import functools

import jax
import jax.numpy as jnp
from jax import lax
from jax.experimental import pallas as pl
from jax.experimental.pallas import tpu as pltpu

_BN_EPS = 1e-5


def _rup(x, m):
    return (x + m - 1) // m * m


def _conv_stats_kernel(x_ref, dw_ref, pw_ref, y_ref, ssum_ref, ssq_ref,
                       xpad_ref, *, H, W, PAD, B):
    HW = H * W
    f32 = jnp.float32
    xpad_ref[:, :PAD] = jnp.zeros_like(xpad_ref[:, :PAD])
    xpad_ref[:, PAD + HW:] = jnp.zeros_like(xpad_ref[:, PAD + HW:])
    ww = lax.broadcasted_iota(jnp.int32, (1, HW), 1) % W
    mleft = (ww > 0).astype(f32)
    mright = (ww < W - 1).astype(f32)

    for s in range(B):
        x = x_ref[s]
        xpad_ref[:, PAD:PAD + HW] = x

        def tap(off):
            return xpad_ref[:, PAD + off:PAD + off + HW]

        def wcol(k):
            return dw_ref[:, k:k + 1]

        gl = wcol(0) * tap(-W - 1) + wcol(3) * tap(-1) + wcol(6) * tap(W - 1)
        gm = wcol(1) * tap(-W) + wcol(4) * x + wcol(7) * tap(W)
        gr = wcol(2) * tap(-W + 1) + wcol(5) * tap(1) + wcol(8) * tap(W + 1)
        z = gl * mleft + gm + gr * mright

        y = jnp.dot(pw_ref[...], z, preferred_element_type=f32)
        y_ref[s] = y.astype(y_ref.dtype)
        ssum_ref[s] = jnp.sum(y, axis=1, keepdims=True)
        ssq_ref[s] = jnp.sum(y * y, axis=1, keepdims=True)


def _bn_relu_kernel(y_ref, ssum_ref, ssq_ref, gamma_ref, beta_ref, o_ref, *,
                    inv_count):
    tot = jnp.sum(ssum_ref[...], axis=0)
    totsq = jnp.sum(ssq_ref[...], axis=0)
    mean = tot * inv_count
    var = jnp.maximum(totsq * inv_count - mean * mean, 0.0)
    scale = gamma_ref[...] * lax.rsqrt(var + _BN_EPS)
    shift = beta_ref[...] - mean * scale
    y = y_ref[...].astype(jnp.float32)
    o_ref[...] = jnp.maximum(y * scale[None] + shift[None], 0.0)


def kernel(x_nchw, dw_weight, pw_weight, gamma, beta):
    N, Cin, H, W = x_nchw.shape
    Cout = pw_weight.shape[0]
    HW = H * W
    f32 = jnp.float32
    PAD = _rup(W + 2, 128)

    x = x_nchw.reshape(N, Cin, HW).astype(f32)
    dww = dw_weight[:, 0, :, :].reshape(Cin, 9).astype(f32)
    pww = pw_weight[:, :, 0, 0].astype(f32)

    b1 = 2 if N % 2 == 0 else 1
    conv_kernel = functools.partial(_conv_stats_kernel, H=H, W=W, PAD=PAD, B=b1)
    y, ssum, ssq = pl.pallas_call(
        conv_kernel,
        out_shape=(jax.ShapeDtypeStruct((N, Cout, HW), jnp.bfloat16),
                   jax.ShapeDtypeStruct((N, Cout, 1), f32),
                   jax.ShapeDtypeStruct((N, Cout, 1), f32)),
        grid_spec=pltpu.PrefetchScalarGridSpec(
            num_scalar_prefetch=0,
            grid=(N // b1,),
            in_specs=[
                pl.BlockSpec((b1, Cin, HW), lambda n: (n, 0, 0)),
                pl.BlockSpec((Cin, 9), lambda n: (0, 0)),
                pl.BlockSpec((Cout, Cin), lambda n: (0, 0)),
            ],
            out_specs=(
                pl.BlockSpec((b1, Cout, HW), lambda n: (n, 0, 0)),
                pl.BlockSpec((b1, Cout, 1), lambda n: (n, 0, 0)),
                pl.BlockSpec((b1, Cout, 1), lambda n: (n, 0, 0)),
            ),
            scratch_shapes=[pltpu.VMEM((Cin, HW + 2 * PAD), f32)],
        ),
        compiler_params=pltpu.CompilerParams(
            dimension_semantics=("parallel",)),
    )(x, dww, pww)

    gamma_c = gamma.astype(f32).reshape(Cout, 1)
    beta_c = beta.astype(f32).reshape(Cout, 1)
    b2 = 2 if N % 2 == 0 else 1
    bn_kernel = functools.partial(_bn_relu_kernel, inv_count=1.0 / float(N * HW))
    out = pl.pallas_call(
        bn_kernel,
        out_shape=jax.ShapeDtypeStruct((N, Cout, HW), f32),
        grid_spec=pltpu.PrefetchScalarGridSpec(
            num_scalar_prefetch=0,
            grid=(N // b2,),
            in_specs=[
                pl.BlockSpec((b2, Cout, HW), lambda n: (n, 0, 0)),
                pl.BlockSpec((N, Cout, 1), lambda n: (0, 0, 0)),
                pl.BlockSpec((N, Cout, 1), lambda n: (0, 0, 0)),
                pl.BlockSpec((Cout, 1), lambda n: (0, 0)),
                pl.BlockSpec((Cout, 1), lambda n: (0, 0)),
            ],
            out_specs=pl.BlockSpec((b2, Cout, HW), lambda n: (n, 0, 0)),
        ),
        compiler_params=pltpu.CompilerParams(
            dimension_semantics=("parallel",)),
    )(y, ssum, ssq, gamma_c, beta_c)

    return out.reshape(N, Cout, H, W)
